# baseline (speedup 1.0000x reference)
import dataclasses
import functools

import jax
import jax.numpy as jnp
from jax import lax
from jax.experimental import pallas as pl
from jax.experimental.pallas import tpu as pltpu
from jax.experimental.pallas import tpu_sc as plsc

EMBED_DIM = 64
PAD_DIM = 128
NUM_CORES = 2
NUM_SUBCORES = 16
NUM_WORKERS = NUM_CORES * NUM_SUBCORES
CHUNK_B = 256
VR = 256


def _compiler_params():
    cp = pltpu.CompilerParams()
    if "needs_layout_passes" in pltpu.CompilerParams.__dataclass_fields__:
        cp = dataclasses.replace(cp, needs_layout_passes=False)
    return cp


@jax.jit
def _sc_relayout(tab_t):
    dim, vocab = tab_t.shape
    n_full = (vocab // VR) * VR
    n_blocks = n_full // VR
    n_iter = -(-n_blocks // NUM_WORKERS)
    tail = vocab - n_full
    mesh = plsc.VectorSubcoreMesh(core_axis_name="c", subcore_axis_name="s")

    @functools.partial(
        pl.kernel,
        mesh=mesh,
        compiler_params=_compiler_params(),
        out_type=jax.ShapeDtypeStruct((vocab // 2, PAD_DIM), jnp.float32),
        scratch_types=[
            pltpu.VMEM((EMBED_DIM, VR), jnp.float32),
            pltpu.VMEM((EMBED_DIM, VR), jnp.float32),
            pltpu.VMEM((VR // 2, PAD_DIM), jnp.float32),
            pltpu.VMEM((VR // 2, PAD_DIM), jnp.float32),
            pltpu.SemaphoreType.DMA,
            pltpu.SemaphoreType.DMA,
            pltpu.SemaphoreType.DMA,
            pltpu.SemaphoreType.DMA,
        ],
    )
    def k(tab_hbm, out_hbm, ti0, ti1, to0, to1, r0, r1, w0, w1):
        tins = (ti0, ti1)
        touts = (to0, to1)
        rsems = (r0, r1)
        wsems = (w0, w1)
        wid = lax.axis_index("s") * NUM_CORES + lax.axis_index("c")
        iota = lax.iota(jnp.int32, 16)
        rots = [(iota + kk) & 15 for kk in range(16)]
        paroff = (iota & 1) << 6
        half = iota >> 1

        def fire(i, slot):
            blk = wid + NUM_WORKERS * i
            pltpu.async_copy(
                tab_hbm.at[:, pl.ds(blk * VR, VR)], tins[slot], rsems[slot]
            )

        def transpose_block(tin, tout, nv):
            @pl.loop(0, nv // 16)
            def _(vb):
                pidx = half + vb * 8
                poff = paroff
                for d0 in range(0, EMBED_DIM, 16):
                    for kk in range(16):
                        didx = rots[kk] + d0
                        v = plsc.load_gather(tin, [didx, iota + vb * 16])
                        plsc.store_scatter(tout, [pidx, didx + poff], v)

        fire(0, 0)

        @pl.loop(0, (n_iter + 1) // 2)
        def _(ci):
            for b2 in range(2):
                i = ci * 2 + b2

                @pl.when(wid + NUM_WORKERS * (i + 1) < n_blocks)
                def _():
                    fire(i + 1, (b2 + 1) % 2)

                @pl.when(wid + NUM_WORKERS * i < n_blocks)
                def _():
                    blk = wid + NUM_WORKERS * i
                    tin = tins[b2]
                    tout = touts[b2]
                    pltpu.make_async_copy(
                        tab_hbm.at[:, pl.ds(blk * VR, VR)], tin, rsems[b2]
                    ).wait()

                    @pl.when(i >= 2)
                    def _():
                        pltpu.make_async_copy(
                            tout, out_hbm.at[pl.ds(0, VR // 2)], wsems[b2]
                        ).wait()

                    transpose_block(tin, tout, VR)

                    pltpu.async_copy(
                        tout,
                        out_hbm.at[pl.ds(blk * (VR // 2), VR // 2)],
                        wsems[b2],
                    )

        for b2 in range(2):
            pltpu.make_async_copy(
                touts[b2], out_hbm.at[pl.ds(0, VR // 2)], wsems[b2]
            ).wait()

        @pl.when(wid == 0)
        def _():
            @pl.loop(0, EMBED_DIM)
            def _(d):
                pltpu.sync_copy(
                    tab_hbm.at[d, pl.ds(n_full, tail)],
                    ti0.at[d, pl.ds(0, tail)],
                )

            transpose_block(ti0, to0, tail)
            pltpu.sync_copy(
                to0.at[pl.ds(0, tail // 2)],
                out_hbm.at[pl.ds(n_full // 2, tail // 2)],
            )

    return k(tab_t)


@functools.partial(jax.jit, static_argnames=("batch", "hist"))
def _sc_lookup(idx_half_t, idx_par_t, table_p, batch, hist):
    bpw = batch // NUM_WORKERS
    sub = bpw // CHUNK_B
    n_chunks = hist * sub
    mesh = plsc.VectorSubcoreMesh(core_axis_name="c", subcore_axis_name="s")

    @functools.partial(
        pl.kernel,
        mesh=mesh,
        compiler_params=_compiler_params(),
        out_type=jax.ShapeDtypeStruct((hist, EMBED_DIM, batch), jnp.float32),
        scratch_types=[
            pltpu.VMEM((CHUNK_B,), jnp.int32),
            pltpu.VMEM((CHUNK_B,), jnp.int32),
            pltpu.VMEM((CHUNK_B,), jnp.int32),
            pltpu.VMEM((CHUNK_B,), jnp.int32),
            pltpu.VMEM((CHUNK_B, PAD_DIM), jnp.float32),
            pltpu.VMEM((CHUNK_B, PAD_DIM), jnp.float32),
            pltpu.VMEM((EMBED_DIM, CHUNK_B), jnp.float32),
            pltpu.VMEM((EMBED_DIM, CHUNK_B), jnp.float32),
            pltpu.SemaphoreType.DMA,
            pltpu.SemaphoreType.DMA,
            pltpu.SemaphoreType.DMA,
            pltpu.SemaphoreType.DMA,
        ],
    )
    def k(
        idxh_hbm, idxp_hbm, tab_hbm, out_hbm,
        ib0, ib1, pb0, pb1, rb0, rb1, ob0, ob1, s0, s1, w0, w1,
    ):
        ibufs = (ib0, ib1)
        pbufs = (pb0, pb1)
        rbufs = (rb0, rb1)
        obufs = (ob0, ob1)
        gsems = (s0, s1)
        wsems = (w0, w1)
        wid = lax.axis_index("s") * NUM_CORES + lax.axis_index("c")
        b_base = wid * bpw
        iota = lax.iota(jnp.int32, 16)
        rots = [(iota + kk) & 15 for kk in range(16)]

        def fire(t, slot):
            h = t // sub
            boff = (t % sub) * CHUNK_B
            pltpu.sync_copy(
                idxh_hbm.at[h, pl.ds(b_base + boff, CHUNK_B)], ibufs[slot]
            )
            pltpu.sync_copy(
                idxp_hbm.at[h, pl.ds(b_base + boff, CHUNK_B)], pbufs[slot]
            )
            pltpu.async_copy(tab_hbm.at[ibufs[slot]], rbufs[slot], gsems[slot])

        fire(0, 0)

        @pl.loop(0, n_chunks // 2)
        def _(c):
            for b in range(2):
                t = c * 2 + b
                h = t // sub
                boff = (t % sub) * CHUNK_B

                @pl.when(t + 1 < n_chunks)
                def _():
                    fire(t + 1, (b + 1) % 2)

                rb = rbufs[b]
                ob = obufs[b]
                pb = pbufs[b]
                pltpu.make_async_copy(
                    tab_hbm.at[ibufs[b]], rb, gsems[b]
                ).wait()

                @pl.when(c >= 1)
                def _():
                    pltpu.make_async_copy(
                        ob,
                        out_hbm.at[0, :, pl.ds(b_base, CHUNK_B)],
                        wsems[b],
                    ).wait()

                @plsc.parallel_loop(0, CHUNK_B // 16, unroll=2)
                def _(jb):
                    jidx = iota + jb * 16
                    pv = pb[pl.ds(jb * 16, 16)]
                    for d0 in range(0, EMBED_DIM, 16):
                        for kk in range(16):
                            didx = rots[kk] + d0
                            v = plsc.load_gather(rb, [jidx, didx + pv])
                            plsc.store_scatter(ob, [didx, jidx], v)

                pltpu.async_copy(
                    ob,
                    out_hbm.at[h, :, pl.ds(b_base + boff, CHUNK_B)],
                    wsems[b],
                )

        for b in range(2):
            pltpu.make_async_copy(
                obufs[b],
                out_hbm.at[0, :, pl.ds(b_base, CHUNK_B)],
                wsems[b],
            ).wait()

    return k(idx_half_t, idx_par_t, table_p)


def kernel(indices, in_embeddings):
    batch, hist = indices.shape
    table_p = _sc_relayout(in_embeddings.T)
    idx_half_t = (indices >> 1).T
    idx_par_t = ((indices & 1) << 6).T
    out3 = _sc_lookup(idx_half_t, idx_par_t, table_p, batch, hist)
    return jnp.transpose(out3, (2, 0, 1))


# speedup vs baseline: 1.1591x; 1.1591x over previous
import dataclasses
import functools

import jax
import jax.numpy as jnp
from jax import lax
from jax.experimental import pallas as pl
from jax.experimental.pallas import tpu as pltpu
from jax.experimental.pallas import tpu_sc as plsc

EMBED_DIM = 64
PAD_DIM = 128
NUM_CORES = 2
NUM_SUBCORES = 16
NUM_WORKERS = NUM_CORES * NUM_SUBCORES
CHUNK_B = 256


@functools.partial(jax.jit, static_argnames=("batch", "hist"))
def _sc_lookup(idx_t, table128, batch, hist):
    bpw = batch // NUM_WORKERS
    sub = bpw // CHUNK_B
    n_chunks = hist * sub
    mesh = plsc.VectorSubcoreMesh(core_axis_name="c", subcore_axis_name="s")
    cp = pltpu.CompilerParams()
    if "needs_layout_passes" in pltpu.CompilerParams.__dataclass_fields__:
        cp = dataclasses.replace(cp, needs_layout_passes=False)

    @functools.partial(
        pl.kernel,
        mesh=mesh,
        compiler_params=cp,
        out_type=jax.ShapeDtypeStruct((hist, EMBED_DIM, batch), jnp.float32),
        scratch_types=[
            pltpu.VMEM((CHUNK_B,), jnp.int32),
            pltpu.VMEM((CHUNK_B,), jnp.int32),
            pltpu.VMEM((CHUNK_B, PAD_DIM), jnp.float32),
            pltpu.VMEM((CHUNK_B, PAD_DIM), jnp.float32),
            pltpu.VMEM((EMBED_DIM, CHUNK_B), jnp.float32),
            pltpu.VMEM((EMBED_DIM, CHUNK_B), jnp.float32),
            pltpu.SemaphoreType.DMA,
            pltpu.SemaphoreType.DMA,
            pltpu.SemaphoreType.DMA,
            pltpu.SemaphoreType.DMA,
        ],
    )
    def k(
        idx_hbm, tab_hbm, out_hbm, ib0, ib1, rb0, rb1, ob0, ob1, s0, s1, w0, w1
    ):
        ibufs = (ib0, ib1)
        rbufs = (rb0, rb1)
        obufs = (ob0, ob1)
        gsems = (s0, s1)
        wsems = (w0, w1)
        wid = lax.axis_index("s") * NUM_CORES + lax.axis_index("c")
        b_base = wid * bpw
        iota = lax.iota(jnp.int32, 16)
        jvecs = [iota + j0 for j0 in range(0, CHUNK_B, 16)]
        rots = [(iota + k) & 15 for k in range(16)]

        def fire(t, slot):
            h = t // sub
            boff = (t % sub) * CHUNK_B
            pltpu.sync_copy(
                idx_hbm.at[h, pl.ds(b_base + boff, CHUNK_B)], ibufs[slot]
            )
            pltpu.async_copy(tab_hbm.at[ibufs[slot]], rbufs[slot], gsems[slot])

        fire(0, 0)

        @pl.loop(0, n_chunks // 2)
        def _(c):
            for b in range(2):
                t = c * 2 + b
                h = t // sub
                boff = (t % sub) * CHUNK_B

                @pl.when(t + 1 < n_chunks)
                def _():
                    fire(t + 1, (b + 1) % 2)

                rb = rbufs[b]
                ob = obufs[b]
                pltpu.make_async_copy(
                    tab_hbm.at[ibufs[b]], rb, gsems[b]
                ).wait()

                @pl.when(c >= 1)
                def _():
                    pltpu.make_async_copy(
                        ob,
                        out_hbm.at[0, :, pl.ds(b_base, CHUNK_B)],
                        wsems[b],
                    ).wait()

                @plsc.parallel_loop(0, CHUNK_B // 16, unroll=2)
                def _(jb):
                    jidx = jvecs[0] + jb * 16
                    for d0 in range(0, EMBED_DIM, 16):
                        for k in range(16):
                            didx = rots[k] + d0
                            v = plsc.load_gather(rb, [jidx, didx])
                            plsc.store_scatter(ob, [didx, jidx], v)

                pltpu.async_copy(
                    ob,
                    out_hbm.at[h, :, pl.ds(b_base + boff, CHUNK_B)],
                    wsems[b],
                )

        for b in range(2):
            pltpu.make_async_copy(
                obufs[b],
                out_hbm.at[0, :, pl.ds(b_base, CHUNK_B)],
                wsems[b],
            ).wait()

    return k(idx_t, table128)


def kernel(indices, in_embeddings):
    batch, hist = indices.shape
    table128 = jnp.pad(in_embeddings, ((0, 0), (0, PAD_DIM - EMBED_DIM)))
    out3 = _sc_lookup(indices.T, table128, batch, hist)
    return jnp.transpose(out3, (2, 0, 1))
